# baseline (speedup 1.0000x reference)
import functools

import jax
import jax.numpy as jnp
from jax import lax
from jax.experimental import pallas as pl
from jax.experimental.pallas import tpu as pltpu
from jax.experimental.pallas import tpu_sc as plsc

_LEN = 2048
_C = 128
_B = 32
_NC = 2
_NS = 16
_PB = 128
_BH = _B // 2
_R = 6
_GLAG = 4


def _pe_table():
    i = jnp.arange(_LEN, dtype=jnp.float32)[:, None]
    j = jnp.arange(_C, dtype=jnp.float32)[None, :]
    val = i / jnp.power(10000.0, j / float(_C))
    even = (jnp.arange(_C)[None, :] % 2) == 0
    return jnp.where(even, jnp.sin(val), jnp.cos(val))


@functools.partial(
    pl.kernel,
    out_type=jax.ShapeDtypeStruct((_B, _LEN, _C), jnp.float32),
    mesh=plsc.VectorSubcoreMesh(core_axis_name="c", subcore_axis_name="s"),
    scratch_types=[
        pltpu.VMEM((_BH, _PB), jnp.int32),
        pltpu.VMEM((3, 2, _PB, _C), jnp.float32),
        pltpu.VMEM_SHARED((_NS, _PB, _C), jnp.float32),
        pltpu.SemaphoreType.DMA((3,)),
        pltpu.SemaphoreType.DMA((3,)),
        pltpu.SemaphoreType.DMA((3,)),
    ],
)
def _embed_sc(x_hbm, w_hbm, pe_hbm, out_hbm, idx_v, buf_v, pe_sh,
              sems_s, sems_g, sems_o):
    c = lax.axis_index("c")
    s = lax.axis_index("s")
    p0 = s * _PB
    b0 = c * _BH
    pltpu.sync_copy(x_hbm.at[pl.ds(b0, _BH), pl.ds(p0, _PB)], idx_v)
    pltpu.sync_copy(pe_hbm.at[pl.ds(p0, _PB)], buf_v.at[0, 0])
    pltpu.sync_copy(buf_v.at[0, 0], pe_sh.at[s])
    pe_slot = pe_sh.at[s]

    _NP = _BH // 2
    seed = [None] * 3
    gat = [None] * 3
    outw = [None] * 3

    def seed_pair(sl):
        return [pltpu.async_copy(pe_slot, buf_v.at[sl, g], sems_s.at[sl])
                for g in range(2)]

    def gather_pair(tp, sl):
        return [pltpu.async_copy(w_hbm.at[idx_v.at[2 * tp + g]],
                                 buf_v.at[sl, g], sems_g.at[sl], add=True)
                for g in range(2)]

    seed[0] = seed_pair(0)
    for tp in range(_NP + 2):
        if tp < _NP:
            sl = tp % 3
            for d in seed[sl]:
                d.wait()
            gat[sl] = gather_pair(tp, sl)
        bw = tp - 1
        if 0 <= bw < _NP:
            sl = bw % 3
            for d in gat[sl]:
                d.wait()
            outw[sl] = pltpu.async_copy(
                buf_v.at[sl], out_hbm.at[pl.ds(b0 + 2 * bw, 2), pl.ds(p0, _PB)],
                sems_o.at[sl])
        bs = tp + 1
        if bs < _NP:
            sl = bs % 3
            if bs >= 3:
                outw[sl].wait()
            seed[sl] = seed_pair(sl)
    for p in range(_NP - 3, _NP):
        outw[p % 3].wait()


def kernel(x, W):
    pe = _pe_table()
    return _embed_sc(x.astype(jnp.int32), W, pe)
